# baseline (speedup 1.0000x reference)
import functools

import jax
import jax.numpy as jnp
from jax import lax
from jax.experimental import pallas as pl
from jax.experimental.pallas import tpu as pltpu
from jax.experimental.pallas import tpu_sc as plsc

BATCH = 16384
D = 32
WIDE = 128
PACK = WIDE // D
NC = 2
NS = 16
L = 16
NW = NC * NS
B_PER_W = BATCH // NW
CHUNK = 128
N_CHUNK = B_PER_W // CHUNK
GROUPS = CHUNK // L


def _bpr_body(urow_h, uoff_h, irow_h, ioff_h, utab_h, itab_h, out_h,
              urow_v, uoff_v, irow_v, ioff_v, utile, itile, out_v,
              sem_u, sem_i):
    c = lax.axis_index("c")
    s = lax.axis_index("s")
    wid = s * NC + c
    base = wid * N_CHUNK

    pltpu.sync_copy(urow_h.at[pl.ds(base, N_CHUNK)], urow_v)
    pltpu.sync_copy(uoff_h.at[pl.ds(base, N_CHUNK)], uoff_v)
    pltpu.sync_copy(irow_h.at[pl.ds(base, N_CHUNK)], irow_v)
    pltpu.sync_copy(ioff_h.at[pl.ds(base, N_CHUNK)], ioff_v)

    iota = jnp.arange(L, dtype=jnp.int32)
    lane = iota

    for j in range(N_CHUNK):
        cp_u = pltpu.async_copy(utab_h.at[urow_v.at[j]], utile, sem_u)
        cp_i = pltpu.async_copy(itab_h.at[irow_v.at[j]], itile, sem_i)
        cp_u.wait()
        cp_i.wait()

        def group_body(g, carry, j=j):
            acc = jnp.zeros((L,), jnp.float32)
            for l in range(L):
                r = g * L + l
                rfull = jnp.full((L,), r, jnp.int32)
                jfull = jnp.full((L,), j, jnp.int32)
                ou = plsc.load_gather(uoff_v, [jfull, rfull])
                oi = plsc.load_gather(ioff_v, [jfull, rfull])
                u0 = plsc.load_gather(utile, [rfull, ou + iota])
                u1 = plsc.load_gather(utile, [rfull, ou + (iota + L)])
                i0 = plsc.load_gather(itile, [rfull, oi + iota])
                i1 = plsc.load_gather(itile, [rfull, oi + (iota + L)])
                v = jnp.sum(u0 * i0 + u1 * i1)
                acc = jnp.where(lane == l, v, acc)
            out_v[pl.ds(j * CHUNK + g * L, L)] = acc
            return carry

        lax.fori_loop(0, GROUPS, group_body, 0)

    pltpu.sync_copy(out_v, out_h.at[pl.ds(wid * B_PER_W, B_PER_W)])


_bpr_sc = functools.partial(
    pl.kernel,
    mesh=plsc.VectorSubcoreMesh(core_axis_name="c", subcore_axis_name="s"),
    out_type=jax.ShapeDtypeStruct((BATCH,), jnp.float32),
    compiler_params=pltpu.CompilerParams(
        needs_layout_passes=False, use_tc_tiling_on_sc=False),
    scratch_types=[
        pltpu.VMEM((N_CHUNK, CHUNK), jnp.int32),
        pltpu.VMEM((N_CHUNK, CHUNK), jnp.int32),
        pltpu.VMEM((N_CHUNK, CHUNK), jnp.int32),
        pltpu.VMEM((N_CHUNK, CHUNK), jnp.int32),
        pltpu.VMEM((CHUNK, WIDE), jnp.float32),
        pltpu.VMEM((CHUNK, WIDE), jnp.float32),
        pltpu.VMEM((B_PER_W,), jnp.float32),
        pltpu.SemaphoreType.DMA,
        pltpu.SemaphoreType.DMA,
    ],
)(_bpr_body)


@jax.jit
def kernel(user_indices, item_indices, user_table, item_table):
    n_rows = BATCH // CHUNK
    urow = (user_indices >> 2).reshape(n_rows, CHUNK)
    uoff = ((user_indices & 3) << 5).reshape(n_rows, CHUNK)
    irow = (item_indices >> 2).reshape(n_rows, CHUNK)
    ioff = ((item_indices & 3) << 5).reshape(n_rows, CHUNK)
    utab = user_table.reshape(-1, WIDE)
    itab = item_table.reshape(-1, WIDE)
    return _bpr_sc(urow, uoff, irow, ioff, utab, itab)


# speedup vs baseline: 1.0018x; 1.0018x over previous
import functools

import jax
import jax.numpy as jnp
from jax import lax
from jax.experimental import pallas as pl
from jax.experimental.pallas import tpu as pltpu
from jax.experimental.pallas import tpu_sc as plsc

BATCH = 16384
D = 32
WIDE = 128
PACK = WIDE // D
NC = 2
NS = 16
L = 16
NW = NC * NS
B_PER_W = BATCH // NW
CHUNK = 128
N_CHUNK = B_PER_W // CHUNK
GROUPS = CHUNK // L


def _bpr_body(urow_h, uoff_h, irow_h, ioff_h, utab_h, itab_h, out_h,
              urow_v, uoff_v, irow_v, ioff_v, utile, itile, out_v,
              sem_u, sem_i):
    c = lax.axis_index("c")
    s = lax.axis_index("s")
    wid = s * NC + c
    base = wid * N_CHUNK

    pltpu.sync_copy(urow_h.at[pl.ds(base, N_CHUNK)], urow_v)
    pltpu.sync_copy(uoff_h.at[pl.ds(base, N_CHUNK)], uoff_v)
    pltpu.sync_copy(irow_h.at[pl.ds(base, N_CHUNK)], irow_v)
    pltpu.sync_copy(ioff_h.at[pl.ds(base, N_CHUNK)], ioff_v)

    iota = jnp.arange(L, dtype=jnp.int32)
    lane = iota

    for j in range(N_CHUNK):
        cp_u = pltpu.async_copy(utab_h.at[urow_v.at[j]], utile, sem_u)
        cp_i = pltpu.async_copy(itab_h.at[irow_v.at[j]], itile, sem_i)
        cp_u.wait()
        cp_i.wait()

        def group_body(g, carry, j=j):
            acc = jnp.zeros((L,), jnp.float32)
            for l in range(L):
                r = g * L + l
                rfull = jnp.full((L,), r, jnp.int32)
                jfull = jnp.full((L,), j, jnp.int32)
                ou = plsc.load_gather(uoff_v, [jfull, rfull])
                oi = plsc.load_gather(ioff_v, [jfull, rfull])
                u0 = plsc.load_gather(utile, [rfull, ou + iota])
                u1 = plsc.load_gather(utile, [rfull, ou + (iota + L)])
                i0 = plsc.load_gather(itile, [rfull, oi + iota])
                i1 = plsc.load_gather(itile, [rfull, oi + (iota + L)])
                v = jnp.sum(u0 * i0 + u1 * i1)
                acc = jnp.where(lane == l, v, acc)
            out_v[pl.ds(j * CHUNK + g * L, L)] = acc
            return carry

        lax.fori_loop(0, GROUPS, group_body, 0)

    pltpu.sync_copy(out_v, out_h.at[pl.ds(wid * B_PER_W, B_PER_W)])


_bpr_sc = functools.partial(
    pl.kernel,
    mesh=plsc.VectorSubcoreMesh(core_axis_name="c", subcore_axis_name="s"),
    out_type=jax.ShapeDtypeStruct((BATCH,), jnp.float32),
    compiler_params=pltpu.CompilerParams(
        needs_layout_passes=False, use_tc_tiling_on_sc=True),
    scratch_types=[
        pltpu.VMEM((N_CHUNK, CHUNK), jnp.int32),
        pltpu.VMEM((N_CHUNK, CHUNK), jnp.int32),
        pltpu.VMEM((N_CHUNK, CHUNK), jnp.int32),
        pltpu.VMEM((N_CHUNK, CHUNK), jnp.int32),
        pltpu.VMEM((CHUNK, WIDE), jnp.float32),
        pltpu.VMEM((CHUNK, WIDE), jnp.float32),
        pltpu.VMEM((B_PER_W,), jnp.float32),
        pltpu.SemaphoreType.DMA,
        pltpu.SemaphoreType.DMA,
    ],
)(_bpr_body)


@jax.jit
def kernel(user_indices, item_indices, user_table, item_table):
    n_rows = BATCH // CHUNK
    urow = (user_indices >> 2).reshape(n_rows, CHUNK)
    uoff = ((user_indices & 3) << 5).reshape(n_rows, CHUNK)
    irow = (item_indices >> 2).reshape(n_rows, CHUNK)
    ioff = ((item_indices & 3) << 5).reshape(n_rows, CHUNK)
    utab = user_table.reshape(-1, WIDE)
    itab = item_table.reshape(-1, WIDE)
    return _bpr_sc(urow, uoff, irow, ioff, utab, itab)


# speedup vs baseline: 1.5126x; 1.5099x over previous
import functools

import jax
import jax.numpy as jnp
from jax import lax
from jax.experimental import pallas as pl
from jax.experimental.pallas import tpu as pltpu
from jax.experimental.pallas import tpu_sc as plsc

BATCH = 16384
D = 32
NC = 2
NS = 16
L = 16
NW = NC * NS
B_PER_W = BATCH // NW
CHUNK = 128
N_CHUNK = B_PER_W // CHUNK
GROUPS = CHUNK // L


def _bpr_body(uidx_h, iidx_h, utab_h, itab_h, out_h,
              uidx_v, iidx_v, utile, itile, out_v, sem_u, sem_i):
    c = lax.axis_index("c")
    s = lax.axis_index("s")
    wid = s * NC + c
    base = wid * B_PER_W

    pltpu.sync_copy(uidx_h.at[pl.ds(base, B_PER_W)], uidx_v)
    pltpu.sync_copy(iidx_h.at[pl.ds(base, B_PER_W)], iidx_v)

    lane = jnp.arange(L, dtype=jnp.int32)

    for j in range(N_CHUNK):
        def fire(g, carry, j=j):
            vu = uidx_v[pl.ds(j * CHUNK + g * L, L)]
            vi = iidx_v[pl.ds(j * CHUNK + g * L, L)]
            for l in range(L):
                k = g * L + l
                pltpu.async_copy(
                    utab_h.at[pl.ds(vu[l], 1)], utile.at[pl.ds(k, 1)], sem_u)
                pltpu.async_copy(
                    itab_h.at[pl.ds(vi[l], 1)], itile.at[pl.ds(k, 1)], sem_i)
            return carry

        lax.fori_loop(0, GROUPS, fire, 0)

        def drain(k, carry):
            pltpu.make_async_copy(
                utab_h.at[pl.ds(0, 1)], utile.at[pl.ds(k, 1)], sem_u).wait()
            pltpu.make_async_copy(
                itab_h.at[pl.ds(0, 1)], itile.at[pl.ds(k, 1)], sem_i).wait()
            return carry

        lax.fori_loop(0, CHUNK, drain, 0)

        def group_body(g, carry, j=j):
            acc = jnp.zeros((L,), jnp.float32)
            for l in range(L):
                r = g * L + l
                u0 = utile[r, pl.ds(0, L)]
                u1 = utile[r, pl.ds(L, L)]
                i0 = itile[r, pl.ds(0, L)]
                i1 = itile[r, pl.ds(L, L)]
                v = jnp.sum(u0 * i0 + u1 * i1)
                acc = jnp.where(lane == l, v, acc)
            out_v[pl.ds(j * CHUNK + g * L, L)] = acc
            return carry

        lax.fori_loop(0, GROUPS, group_body, 0)

    pltpu.sync_copy(out_v, out_h.at[pl.ds(base, B_PER_W)])


_bpr_sc = functools.partial(
    pl.kernel,
    mesh=plsc.VectorSubcoreMesh(core_axis_name="c", subcore_axis_name="s"),
    out_type=jax.ShapeDtypeStruct((BATCH,), jnp.float32),
    compiler_params=pltpu.CompilerParams(
        needs_layout_passes=False, use_tc_tiling_on_sc=True),
    scratch_types=[
        pltpu.VMEM((B_PER_W,), jnp.int32),
        pltpu.VMEM((B_PER_W,), jnp.int32),
        pltpu.VMEM((CHUNK, D), jnp.float32),
        pltpu.VMEM((CHUNK, D), jnp.float32),
        pltpu.VMEM((B_PER_W,), jnp.float32),
        pltpu.SemaphoreType.DMA,
        pltpu.SemaphoreType.DMA,
    ],
)(_bpr_body)


@jax.jit
def kernel(user_indices, item_indices, user_table, item_table):
    return _bpr_sc(user_indices, item_indices, user_table, item_table)
